# baseline (speedup 1.0000x reference)
import functools

import jax
import jax.numpy as jnp
from jax.experimental import pallas as pl
from jax.experimental.pallas import tpu as pltpu

_K = 100
_BM = 512


def _tree_count(m_bool):
    s = jnp.where(m_bool, jnp.bfloat16(1), jnp.bfloat16(0))
    while s.shape[1] > 128:
        h = s.shape[1] // 2
        s = s[:, :h] + s[:, h:]
    return jnp.sum(s.astype(jnp.float32), axis=1, keepdims=True)


def _body(x_ref, we_ref, be_ref, wd_ref, bd_ref, o_ref, enc_ref):
    s = pl.program_id(0)
    nb = pl.num_programs(0) - 1
    cur = jax.lax.rem(s, 2)
    prv = 1 - cur

    def decode_chunk(j, src):
        js = pl.multiple_of(j * 256, 256)
        o_ref[:, pl.ds(js, 256)] = (
            jnp.dot(
                src, wd_ref[:, pl.ds(js, 256)], preferred_element_type=jnp.float32
            )
            + bd_ref[:, pl.ds(js, 256)]
        )

    @pl.when(s < nb)
    def _compute():
        xm = (x_ref[...] - bd_ref[...]).astype(jnp.bfloat16)
        pre = jnp.dot(xm, we_ref[...], preferred_element_type=jnp.float32)
        y = jnp.maximum(pre + be_ref[...], 0.0)
        bits = jax.lax.bitcast_convert_type(y, jnp.int32)
        bm = y.shape[0]
        kf = jnp.float32(_K)

        y16 = jax.lax.bitcast_convert_type(
            jnp.bitwise_and(bits, jnp.int32(-65536)), jnp.float32
        ).astype(jnp.bfloat16)
        lo = (jnp.bitwise_xor(bits, 0x8000) & 0xFFFF).astype(jnp.int16)

        def step1(i, t):
            cand = jnp.bitwise_or(t, jax.lax.shift_left(1, 14 - i))
            cand_b = jax.lax.bitcast_convert_type(
                cand.astype(jnp.int16), jnp.bfloat16
            )
            cnt = _tree_count(y16 >= cand_b)
            return jnp.where(cnt >= kf, cand, t)

        t1 = jax.lax.fori_loop(0, 15, step1, jnp.zeros((bm, 1), jnp.int32))
        t1_b = jax.lax.bitcast_convert_type(t1.astype(jnp.int16), jnp.bfloat16)
        n_gt = _tree_count(y16 > t1_b)
        meq = y16 == t1_b

        def step2(i, t):
            cand = jnp.bitwise_or(t, jax.lax.shift_left(1, 15 - i))
            cand16 = jnp.bitwise_xor(cand, 0x8000).astype(jnp.int16)
            cnt = n_gt + _tree_count((lo >= cand16) & meq)

            @pl.when(jnp.logical_and(s > 0, i >= 12))
            def _():
                decode_chunk(i - 12, enc_ref[prv])

            return jnp.where(cnt >= kf, cand, t)

        u = jax.lax.fori_loop(0, 16, step2, jnp.zeros((bm, 1), jnp.int32))
        thr = jnp.bitwise_or(jax.lax.shift_left(t1, 16), u)
        enc_ref[cur] = jnp.where(bits >= thr, y, 0.0).astype(jnp.bfloat16)

    @pl.when(s == nb)
    def _tail():
        for j in range(4):
            decode_chunk(j, enc_ref[prv])


@jax.jit
def kernel(x, W_enc, b_enc, W_dec, b_dec):
    B, d_in = x.shape
    d_sae = W_enc.shape[1]
    nb = B // _BM
    be = b_enc.reshape(1, d_sae)
    bd = b_dec.reshape(1, d_in)
    return pl.pallas_call(
        _body,
        grid=(nb + 1,),
        in_specs=[
            pl.BlockSpec((_BM, d_in), lambda i: (jnp.minimum(i, nb - 1), 0)),
            pl.BlockSpec((d_in, d_sae), lambda i: (0, 0)),
            pl.BlockSpec((1, d_sae), lambda i: (0, 0)),
            pl.BlockSpec((d_sae, d_in), lambda i: (0, 0)),
            pl.BlockSpec((1, d_in), lambda i: (0, 0)),
        ],
        out_specs=pl.BlockSpec(
            (_BM, d_in), lambda i: (jnp.maximum(i - 1, 0), 0)
        ),
        out_shape=jax.ShapeDtypeStruct((B, d_in), jnp.float32),
        scratch_shapes=[pltpu.VMEM((2, _BM, d_sae), jnp.bfloat16)],
    )(x, W_enc.astype(jnp.bfloat16), be, W_dec.astype(jnp.bfloat16), bd)


# speedup vs baseline: 1.1479x; 1.1479x over previous
import functools

import jax
import jax.numpy as jnp
from jax.experimental import pallas as pl
from jax.experimental.pallas import tpu as pltpu

_K = 100
_BM = 512


def _tree_count(m_bool):
    s = jnp.where(m_bool, jnp.bfloat16(1), jnp.bfloat16(0))
    while s.shape[1] > 128:
        h = s.shape[1] // 2
        s = s[:, :h] + s[:, h:]
    return jnp.sum(s.astype(jnp.float32), axis=1, keepdims=True)


def _body(x_ref, we_ref, be_ref, wd_ref, bd_ref, o_ref, enc_ref):
    s = pl.program_id(0)
    nb = pl.num_programs(0) - 1
    cur = jax.lax.rem(s, 2)
    prv = 1 - cur

    def decode_chunk(j, src):
        js = pl.multiple_of(j * 256, 256)
        o_ref[:, pl.ds(js, 256)] = (
            jnp.dot(
                src, wd_ref[:, pl.ds(js, 256)], preferred_element_type=jnp.float32
            )
            + bd_ref[:, pl.ds(js, 256)]
        )

    @pl.when(s < nb)
    def _compute():
        xm = (x_ref[...] - bd_ref[...]).astype(jnp.bfloat16)
        pre = jnp.dot(xm, we_ref[...], preferred_element_type=jnp.float32)
        y = jnp.maximum(pre + be_ref[...], 0.0)
        bits = jax.lax.bitcast_convert_type(y, jnp.int32)
        bm = y.shape[0]
        kf = jnp.float32(_K)

        y16 = jax.lax.bitcast_convert_type(
            jnp.bitwise_and(bits, jnp.int32(-65536)), jnp.float32
        ).astype(jnp.bfloat16)
        lo = (jnp.bitwise_xor(bits, 0x8000) & 0xFFFF).astype(jnp.int16)

        def step1(i, t):
            cand = jnp.bitwise_or(t, jax.lax.shift_left(1, 14 - i))
            cand_b = jax.lax.bitcast_convert_type(
                cand.astype(jnp.int16), jnp.bfloat16
            )
            cnt = _tree_count(y16 >= cand_b)
            return jnp.where(cnt >= kf, cand, t)

        t1 = jax.lax.fori_loop(0, 15, step1, jnp.zeros((bm, 1), jnp.int32))
        t1_b = jax.lax.bitcast_convert_type(t1.astype(jnp.int16), jnp.bfloat16)
        n_gt = _tree_count(y16 > t1_b)
        meq = y16 == t1_b

        def step2(i, t):
            cand = jnp.bitwise_or(t, jax.lax.shift_left(1, 15 - i))
            cand16 = jnp.bitwise_xor(cand, 0x8000).astype(jnp.int16)
            cnt = n_gt + _tree_count((lo >= cand16) & meq)
            return jnp.where(cnt >= kf, cand, t)

        u = jax.lax.fori_loop(0, 12, step2, jnp.zeros((bm, 1), jnp.int32))
        enc_prev = enc_ref[prv]
        for i in range(12, 16):
            decode_chunk(i - 12, enc_prev)
            u = step2(i, u)
        thr = jnp.bitwise_or(jax.lax.shift_left(t1, 16), u)
        enc_ref[cur] = jnp.where(bits >= thr, y, 0.0).astype(jnp.bfloat16)

    @pl.when(s == nb)
    def _tail():
        for j in range(4):
            decode_chunk(j, enc_ref[prv])


@jax.jit
def kernel(x, W_enc, b_enc, W_dec, b_dec):
    B, d_in = x.shape
    d_sae = W_enc.shape[1]
    nb = B // _BM
    be = b_enc.reshape(1, d_sae)
    bd = b_dec.reshape(1, d_in)
    return pl.pallas_call(
        _body,
        grid=(nb + 1,),
        in_specs=[
            pl.BlockSpec((_BM, d_in), lambda i: (jnp.minimum(i, nb - 1), 0)),
            pl.BlockSpec((d_in, d_sae), lambda i: (0, 0)),
            pl.BlockSpec((1, d_sae), lambda i: (0, 0)),
            pl.BlockSpec((d_sae, d_in), lambda i: (0, 0)),
            pl.BlockSpec((1, d_in), lambda i: (0, 0)),
        ],
        out_specs=pl.BlockSpec(
            (_BM, d_in), lambda i: (jnp.maximum(i - 1, 0), 0)
        ),
        out_shape=jax.ShapeDtypeStruct((B, d_in), jnp.float32),
        scratch_shapes=[pltpu.VMEM((2, _BM, d_sae), jnp.bfloat16)],
    )(x, W_enc.astype(jnp.bfloat16), be, W_dec.astype(jnp.bfloat16), bd)
